# Initial kernel scaffold; baseline (speedup 1.0000x reference)
import jax
import jax.numpy as jnp
from jax.experimental import pallas as pl


def kernel(edge_attr, W0, W1):
    raise NotImplementedError("write your pallas kernel here")


# speedup vs baseline: 2.7968x; 2.7968x over previous
import functools

import jax
import jax.numpy as jnp
from jax import lax
from jax.experimental import pallas as pl
from jax.experimental.pallas import tpu as pltpu
from jax.experimental.pallas import tpu_sc as plsc

E = 3_200_000
EMB = 16
OUT_D = 32
NC = 2
NS = 16
L = 16
NW = NC * NS
E_PER_W = E // NW
CHUNK = 2000
NCHUNK = E_PER_W // CHUNK
GROUPS = CHUNK // L

_mesh = plsc.VectorSubcoreMesh(core_axis_name="c", subcore_axis_name="s")


@functools.partial(
    pl.kernel,
    mesh=_mesh,
    compiler_params=pltpu.CompilerParams(
        needs_layout_passes=False, use_tc_tiling_on_sc=False
    ),
    out_type=jax.ShapeDtypeStruct((E, OUT_D), jnp.float32),
    scratch_types=[
        pltpu.VMEM((4, EMB), jnp.float32),
        pltpu.VMEM((4, EMB), jnp.float32),
        pltpu.VMEM((CHUNK, 2), jnp.int32),
        pltpu.VMEM((CHUNK, OUT_D), jnp.float32),
    ],
)
def _edge_encode(edge_hbm, w0_hbm, w1_hbm, out_hbm, w0_v, w1_v, idx_v, out_v):
    wid = lax.axis_index("s") * NC + lax.axis_index("c")
    base = wid * E_PER_W
    pltpu.sync_copy(w0_hbm, w0_v)
    pltpu.sync_copy(w1_hbm, w1_v)

    iota = lax.iota(jnp.int32, L)
    zeros = jnp.zeros((L,), jnp.int32)
    ones = jnp.ones((L,), jnp.int32)

    def chunk_body(ci, carry):
        start = base + ci * CHUNK
        pltpu.sync_copy(edge_hbm.at[pl.ds(start, CHUNK)], idx_v)

        def grp(gi, c2):
            e_loc = iota + gi * L
            a0 = plsc.load_gather(idx_v, [e_loc, zeros])
            a1 = plsc.load_gather(idx_v, [e_loc, ones])
            for c in range(EMB):
                cc = jnp.full((L,), c, jnp.int32)
                v0 = plsc.load_gather(w0_v, [a0, cc])
                plsc.store_scatter(out_v, [e_loc, cc], v0)
                v1 = plsc.load_gather(w1_v, [a1, cc])
                plsc.store_scatter(out_v, [e_loc, cc + EMB], v1)
            return c2

        lax.fori_loop(0, GROUPS, grp, 0)
        pltpu.sync_copy(out_v, out_hbm.at[pl.ds(start, CHUNK)])
        return carry

    lax.fori_loop(0, NCHUNK, chunk_body, 0)


def kernel(edge_attr, W0, W1):
    return _edge_encode(edge_attr, W0, W1)
